# baseline (speedup 1.0000x reference)
import functools

import jax
import jax.numpy as jnp
from jax import lax
from jax.experimental import pallas as pl
from jax.experimental.pallas import tpu as pltpu
from jax.experimental.pallas import tpu_sc as plsc

_NC = 2
_NS = 16
_NW = _NC * _NS
_BPW = 3200 // _NW
_NP = 4


def _emb_lookup(x3, table, idm):
    mesh = plsc.VectorSubcoreMesh(core_axis_name="c", subcore_axis_name="s")
    S = _BPW

    @functools.partial(
        pl.kernel,
        mesh=mesh,
        out_type=jax.ShapeDtypeStruct((200, 4, 128, 8, 128), jnp.float32),
        compiler_params=pltpu.CompilerParams(
            use_tc_tiling_on_sc=False, needs_layout_passes=False),
        scratch_types=[
            pltpu.VMEM((1024,), jnp.int32), pltpu.VMEM((1024,), jnp.int32),
            pltpu.VMEM((1024,), jnp.int32), pltpu.VMEM((1024,), jnp.int32),
            pltpu.VMEM((1024, 32), jnp.float32), pltpu.VMEM((1024, 32), jnp.float32),
            pltpu.VMEM((4, 8, 128), jnp.float32), pltpu.VMEM((4, 8, 128), jnp.float32),
            pltpu.VMEM((4, 8, 128), jnp.float32), pltpu.VMEM((4, 8, 128), jnp.float32),
            pltpu.SemaphoreType.DMA, pltpu.SemaphoreType.DMA,
            pltpu.SemaphoreType.DMA, pltpu.SemaphoreType.DMA,
            pltpu.SemaphoreType.DMA, pltpu.SemaphoreType.DMA,
            pltpu.SemaphoreType.DMA, pltpu.SemaphoreType.DMA,
            pltpu.SemaphoreType.DMA, pltpu.SemaphoreType.DMA,
        ],
    )
    def emb_kernel(x_hbm, tab_hbm, map_hbm, out_hbm,
                   xv0, xv1, mv0, mv1, rw0, rw1, ob0, ob1, ob2, ob3,
                   sx0, sx1, sm0, sm1, st0, st1, so0, so1, so2, so3):
        wid = lax.axis_index("s") * _NC + lax.axis_index("c")
        blk0 = wid * S
        xv = (xv0, xv1)
        mv = (mv0, mv1)
        rw = (rw0, rw1)
        ob = (ob0, ob1, ob2, ob3)
        sx = (sx0, sx1)
        sm = (sm0, sm1)
        st = (st0, st1)
        so = (so0, so1, so2, so3)
        iota16 = jnp.arange(16, dtype=jnp.int32)

        def blk(t):
            g = blk0 + t
            return g >> 7, g & 127

        def start_x(t, b):
            ts, tb = blk(t)
            pltpu.async_copy(x_hbm.at[ts, tb], xv[b], sx[b])

        def wait_x(b):
            pltpu.make_async_copy(x_hbm.at[0, 0], xv[b], sx[b]).wait()

        def start_map(b):
            pltpu.async_copy(map_hbm.at[xv[b]], mv[b], sm[b])

        def wait_map(b):
            pltpu.make_async_copy(map_hbm.at[xv[b]], mv[b], sm[b]).wait()

        def start_tab(b):
            pltpu.async_copy(tab_hbm.at[mv[b]], rw[b], st[b])

        def wait_tab(b):
            pltpu.make_async_copy(tab_hbm.at[mv[b]], rw[b], st[b]).wait()

        def start_ob(t, s_in, p):
            ts, tb = blk(t)
            pltpu.async_copy(ob[p], out_hbm.at[ts * 8 + s_in, :, tb], so[p])

        def wait_ob(p):
            pltpu.make_async_copy(ob[p], out_hbm.at[0, :, 0], so[p]).wait()

        def transpose_one(b, s_in, p):
            rb = s_in * 128

            def kbody(k, _):
                ir = iota16 + (rb + k * 16)
                for d in range(32):
                    v = plsc.load_gather(
                        rw[b], [ir, jnp.full((16,), d, jnp.int32)])
                    ob[p][d // 8, d % 8, pl.ds(k * 16, 16)] = v
                return 0

            lax.fori_loop(0, 8, kbody, 0)

        start_x(0, 0)
        start_x(1, 1)
        wait_x(0)
        start_map(0)

        def stage(t, b):
            nb = 1 - b

            @pl.when(t < S)
            def _():
                wait_map(b)
                start_tab(b)

            @pl.when(jnp.logical_and(t >= 1, t <= S))
            def _():
                wait_tab(nb)

            @pl.when(t + 1 < S)
            def _():
                wait_x(nb)
                start_map(nb)

            @pl.when(t + 2 < S)
            def _():
                start_x(t + 2, b)

            @pl.when(jnp.logical_and(t >= 1, t <= S))
            def _():
                for s_in in range(8):
                    p = s_in % _NP
                    if s_in >= _NP:
                        wait_ob(p)
                    else:
                        @pl.when(t >= 2)
                        def _():
                            wait_ob(p)
                    transpose_one(nb, s_in, p)
                    start_ob(t - 1, s_in, p)

        def body(i, _):
            t0 = i * 2
            stage(t0, 0)
            stage(t0 + 1, 1)
            return 0

        lax.fori_loop(0, (S + 2) // 2, body, 0)
        for p in range(_NP):
            wait_ob(p)

    return emb_kernel(x3, table, idm)


def kernel(x, table, id_mapper):
    x3 = (x.astype(jnp.int32)
           .reshape(128, 128, 25, 8)
           .transpose(2, 0, 3, 1)
           .reshape(25, 128, 1024))
    idm = id_mapper.astype(jnp.int32)
    y5 = _emb_lookup(x3, table, idm)
    return y5.transpose(2, 4, 0, 1, 3).reshape(16384, 200, 32)


# speedup vs baseline: 1.3906x; 1.3906x over previous
import functools

import jax
import jax.numpy as jnp
from jax import lax
from jax.experimental import pallas as pl
from jax.experimental.pallas import tpu as pltpu
from jax.experimental.pallas import tpu_sc as plsc

_NC = 2
_NS = 16
_NW = _NC * _NS
_BPW = 3200 // _NW
_NP = 4


def _emb_lookup(x3, table, idm):
    mesh = plsc.VectorSubcoreMesh(core_axis_name="c", subcore_axis_name="s")
    S = _BPW

    @functools.partial(
        pl.kernel,
        mesh=mesh,
        out_type=jax.ShapeDtypeStruct((200, 4, 128, 8, 128), jnp.float32),
        compiler_params=pltpu.CompilerParams(
            use_tc_tiling_on_sc=False, needs_layout_passes=False,
            disable_bounds_checks=True),
        scratch_types=[
            pltpu.VMEM((1024,), jnp.int32), pltpu.VMEM((1024,), jnp.int32),
            pltpu.VMEM((1024,), jnp.int32), pltpu.VMEM((1024,), jnp.int32),
            pltpu.VMEM((1024, 32), jnp.float32), pltpu.VMEM((1024, 32), jnp.float32),
            pltpu.VMEM((4, 8, 128), jnp.float32), pltpu.VMEM((4, 8, 128), jnp.float32),
            pltpu.VMEM((4, 8, 128), jnp.float32), pltpu.VMEM((4, 8, 128), jnp.float32),
            pltpu.SemaphoreType.DMA, pltpu.SemaphoreType.DMA,
            pltpu.SemaphoreType.DMA, pltpu.SemaphoreType.DMA,
            pltpu.SemaphoreType.DMA, pltpu.SemaphoreType.DMA,
            pltpu.SemaphoreType.DMA, pltpu.SemaphoreType.DMA,
            pltpu.SemaphoreType.DMA, pltpu.SemaphoreType.DMA,
        ],
    )
    def emb_kernel(x_hbm, tab_hbm, map_hbm, out_hbm,
                   xv0, xv1, mv0, mv1, rw0, rw1, ob0, ob1, ob2, ob3,
                   sx0, sx1, sm0, sm1, st0, st1, so0, so1, so2, so3):
        wid = lax.axis_index("s") * _NC + lax.axis_index("c")
        blk0 = wid * S
        xv = (xv0, xv1)
        mv = (mv0, mv1)
        rw = (rw0, rw1)
        ob = (ob0, ob1, ob2, ob3)
        sx = (sx0, sx1)
        sm = (sm0, sm1)
        st = (st0, st1)
        so = (so0, so1, so2, so3)
        iota16 = jnp.arange(16, dtype=jnp.int32)

        def blk(t):
            g = blk0 + t
            return g >> 7, g & 127

        def start_x(t, b):
            ts, tb = blk(t)
            pltpu.async_copy(x_hbm.at[ts, tb], xv[b], sx[b])

        def wait_x(b):
            pltpu.make_async_copy(x_hbm.at[0, 0], xv[b], sx[b]).wait()

        def start_map(b):
            pltpu.async_copy(map_hbm.at[xv[b]], mv[b], sm[b])

        def wait_map(b):
            pltpu.make_async_copy(map_hbm.at[xv[b]], mv[b], sm[b]).wait()

        def start_tab(b):
            pltpu.async_copy(tab_hbm.at[mv[b]], rw[b], st[b])

        def wait_tab(b):
            pltpu.make_async_copy(tab_hbm.at[mv[b]], rw[b], st[b]).wait()

        def start_ob(t, s_in, p):
            ts, tb = blk(t)
            pltpu.async_copy(ob[p], out_hbm.at[ts * 8 + s_in, :, tb], so[p])

        def wait_ob(p):
            pltpu.make_async_copy(ob[p], out_hbm.at[0, :, 0], so[p]).wait()

        def transpose_one(b, s_in, p):
            rb = s_in * 128

            @plsc.parallel_loop(0, 8, unroll=2)
            def _(k):
                ir = iota16 + (rb + k * 16)
                for d in range(32):
                    v = plsc.load_gather(
                        rw[b], [ir, jnp.full((16,), d, jnp.int32)])
                    ob[p][d // 8, d % 8, pl.ds(k * 16, 16)] = v

        start_x(0, 0)
        start_x(1, 1)
        wait_x(0)
        start_map(0)

        def stage(t, b):
            nb = 1 - b

            @pl.when(t < S)
            def _():
                wait_map(b)
                start_tab(b)

            @pl.when(jnp.logical_and(t >= 1, t <= S))
            def _():
                wait_tab(nb)

            @pl.when(t + 1 < S)
            def _():
                wait_x(nb)
                start_map(nb)

            @pl.when(t + 2 < S)
            def _():
                start_x(t + 2, b)

            @pl.when(jnp.logical_and(t >= 1, t <= S))
            def _():
                for s_in in range(8):
                    p = s_in % _NP
                    if s_in >= _NP:
                        wait_ob(p)
                    else:
                        @pl.when(t >= 2)
                        def _():
                            wait_ob(p)
                    transpose_one(nb, s_in, p)
                    start_ob(t - 1, s_in, p)

        def body(i, _):
            t0 = i * 2
            stage(t0, 0)
            stage(t0 + 1, 1)
            return 0

        lax.fori_loop(0, (S + 2) // 2, body, 0)
        for p in range(_NP):
            wait_ob(p)

    return emb_kernel(x3, table, idm)


def kernel(x, table, id_mapper):
    x3 = (x.astype(jnp.int32)
           .reshape(128, 128, 25, 8)
           .transpose(2, 0, 3, 1)
           .reshape(25, 128, 1024))
    idm = id_mapper.astype(jnp.int32)
    y5 = _emb_lookup(x3, table, idm)
    return y5.transpose(2, 4, 0, 1, 3).reshape(16384, 200, 32)
